# baseline (speedup 1.0000x reference)
import functools

import jax
import jax.numpy as jnp
from jax import lax
from jax.experimental import pallas as pl
from jax.experimental.pallas import tpu as pltpu
from jax.experimental.pallas import tpu_sc as plsc

_NUM_FIELDS = 26
_VOCAB = 100000
_Q = 25088
_T0 = _Q
_T1 = _Q
_T2 = _Q
_T3 = _VOCAB - 3 * _Q
_OFF1 = _Q
_OFF2 = 2 * _Q
_OFF3 = 3 * _Q
_EMBED_DIM = 32
_BATCH = 4096

_ROWS = _NUM_FIELDS * _EMBED_DIM
_NUM_CORES = 2
_NUM_SUBCORES = 16
_LANES = 16
_UNROLL = 8
_BVECS = _BATCH // _LANES


def _make_gather():
    mesh = plsc.VectorSubcoreMesh(core_axis_name="c", subcore_axis_name="s")

    @functools.partial(
        pl.kernel,
        mesh=mesh,
        out_type=jax.ShapeDtypeStruct((_ROWS, _BATCH), jnp.float32),
        scratch_types=[
            pltpu.VMEM((_T0,), jnp.float32),
            pltpu.VMEM((_T1,), jnp.float32),
            pltpu.VMEM((_T2,), jnp.float32),
            pltpu.VMEM((_T3,), jnp.float32),
            pltpu.SemaphoreType.DMA,
            pltpu.VMEM((_BATCH,), jnp.int32),
            pltpu.VMEM((_BATCH,), jnp.int32),
            pltpu.VMEM((_BATCH,), jnp.float32),
            pltpu.VMEM((_BATCH,), jnp.float32),
            pltpu.SemaphoreType.DMA,
            pltpu.SemaphoreType.DMA,
            pltpu.SemaphoreType.DMA,
            pltpu.SemaphoreType.DMA,
            pltpu.SemaphoreType.DMA,
            pltpu.SemaphoreType.DMA,
            pltpu.SemaphoreType.DMA,
        ],
        compiler_params=pltpu.CompilerParams(needs_layout_passes=False),
    )
    def gather_k(tab_hbm, idx_hbm, out_hbm,
                 buf0, buf1, buf2, buf3, sem3, idx0, idx1, outv0, outv1,
                 sem0, sem1, sem2, sem_i0, sem_i1, sem_o0, sem_o1):
        w = lax.axis_index("c") * _NUM_SUBCORES + lax.axis_index("s")

        def third(r, off, n, buf, sem):
            return pltpu.make_async_copy(
                tab_hbm.at[r].at[pl.ds(off, n)], buf, sem)

        def idx_dma(k, buf, sem):
            return pltpu.make_async_copy(idx_hbm.at[k], buf, sem)

        def out_dma(r, buf, sem):
            return pltpu.make_async_copy(buf, out_hbm.at[r], sem)

        iota16 = lax.iota(jnp.int32, _LANES)

        def pass0(idx_v, out_v):
            def body(j, c2):
                for u in range(_UNROLL):
                    sl = pl.ds((j * _UNROLL + u) * _LANES, _LANES)
                    iv = idx_v[sl]
                    m = iv < _T0
                    out_v[sl] = plsc.load_gather(buf0, [iv], mask=m)
                return c2

            lax.fori_loop(0, _BVECS // _UNROLL, body, 0)

        def mid_pass(buf, off, size):
            def run(idx_v, out_v):
                def body(j, c2):
                    for u in range(_UNROLL):
                        v = j * _UNROLL + u
                        sl = pl.ds(v * _LANES, _LANES)
                        d = idx_v[sl] - off
                        m = d.astype(jnp.uint32) < jnp.uint32(size)
                        g = plsc.load_gather(buf, [d], mask=m)
                        pos = v * _LANES + iota16
                        plsc.store_scatter(out_v, [pos], g, mask=m)
                    return c2

                lax.fori_loop(0, _BVECS // _UNROLL, body, 0)

            return run

        def last_pass(buf, off):
            def run(idx_v, out_v):
                def body(j, c2):
                    for u in range(_UNROLL):
                        v = j * _UNROLL + u
                        sl = pl.ds(v * _LANES, _LANES)
                        d = idx_v[sl] - off
                        m = d >= 0
                        g = plsc.load_gather(buf, [d], mask=m)
                        pos = v * _LANES + iota16
                        plsc.store_scatter(out_v, [pos], g, mask=m)
                    return c2

                lax.fori_loop(0, _BVECS // _UNROLL, body, 0)

            return run

        pass1 = mid_pass(buf1, _OFF1, _T1)
        pass2 = mid_pass(buf2, _OFF2, _T2)
        pass3 = last_pass(buf3, _OFF3)

        def field(r, idx_v, out_v, has_next):
            for off, n, buf, sem, p in ((0, _T0, buf0, sem0, pass0),
                                        (_OFF1, _T1, buf1, sem1, pass1),
                                        (_OFF2, _T2, buf2, sem2, pass2),
                                        (_OFF3, _T3, buf3, sem3, pass3)):
                pltpu.make_async_copy(
                    tab_hbm.at[r].at[pl.ds(off, n)], buf, sem).wait()
                p(idx_v, out_v)

                @pl.when(has_next)
                def _(off=off, n=n, buf=buf, sem=sem):
                    third(r + _EMBED_DIM, off, n, buf, sem).start()

        third(w, 0, _T0, buf0, sem0).start()
        third(w, _OFF1, _T1, buf1, sem1).start()
        third(w, _OFF2, _T2, buf2, sem2).start()
        third(w, _OFF3, _T3, buf3, sem3).start()
        pltpu.sync_copy(idx_hbm.at[0], idx0)

        def field_pair(m, carry):
            k0 = m * 2
            k1 = k0 + 1
            r0 = k0 * _EMBED_DIM + w
            r1 = r0 + _EMBED_DIM

            @pl.when(m > 0)
            def _():
                out_dma(r0, outv0, sem_o0).wait()

            idx_dma(k1, idx1, sem_i1).start()
            field(r0, idx0, outv0, k1 < _NUM_FIELDS)
            out_dma(r0, outv0, sem_o0).start()

            @pl.when(m > 0)
            def _():
                out_dma(r1, outv1, sem_o1).wait()

            @pl.when(k1 + 1 < _NUM_FIELDS)
            def _():
                idx_dma(k1 + 1, idx0, sem_i0).start()

            idx_dma(k1, idx1, sem_i1).wait()
            field(r1, idx1, outv1, k1 + 1 < _NUM_FIELDS)
            out_dma(r1, outv1, sem_o1).start()

            @pl.when(k1 + 1 < _NUM_FIELDS)
            def _():
                idx_dma(k1 + 1, idx0, sem_i0).wait()

            return carry

        lax.fori_loop(0, _NUM_FIELDS // 2, field_pair, 0)
        out_dma(_ROWS - 2 * _EMBED_DIM + w, outv0, sem_o0).wait()
        out_dma(_ROWS - _EMBED_DIM + w, outv1, sem_o1).wait()

    return gather_k


_gather = _make_gather()


def kernel(x_cat, tables):
    tab2d = jnp.transpose(tables, (0, 2, 1)).reshape(_ROWS, _VOCAB)
    xt = jnp.transpose(x_cat.astype(jnp.int32))
    out_t = _gather(tab2d, xt)
    return jnp.transpose(out_t)


# speedup vs baseline: 1.6906x; 1.6906x over previous
import functools

import jax
import jax.numpy as jnp
from jax import lax
from jax.experimental import pallas as pl
from jax.experimental.pallas import tpu as pltpu
from jax.experimental.pallas import tpu_sc as plsc

_NUM_FIELDS = 26
_VOCAB = 100000
_Q = 25088
_T0 = _Q
_T1 = _Q
_T2 = _Q
_T3 = _VOCAB - 3 * _Q
_OFF1 = _Q
_OFF2 = 2 * _Q
_OFF3 = 3 * _Q
_EMBED_DIM = 32
_BATCH = 4096

_ROWS = _NUM_FIELDS * _EMBED_DIM
_NUM_CORES = 2
_NUM_SUBCORES = 16
_LANES = 16
_UNROLL = 8
_BVECS = _BATCH // _LANES


def _make_gather():
    mesh = plsc.VectorSubcoreMesh(core_axis_name="c", subcore_axis_name="s")

    @functools.partial(
        pl.kernel,
        mesh=mesh,
        out_type=jax.ShapeDtypeStruct((_ROWS, _BATCH), jnp.float32),
        scratch_types=[
            pltpu.VMEM((_T0,), jnp.float32),
            pltpu.VMEM((_T1,), jnp.float32),
            pltpu.VMEM((_T2,), jnp.float32),
            pltpu.VMEM((_T3,), jnp.float32),
            pltpu.SemaphoreType.DMA,
            pltpu.VMEM((_BATCH,), jnp.int32),
            pltpu.VMEM((_BATCH,), jnp.int32),
            pltpu.VMEM((_BATCH,), jnp.float32),
            pltpu.VMEM((_BATCH,), jnp.float32),
            pltpu.SemaphoreType.DMA,
            pltpu.SemaphoreType.DMA,
            pltpu.SemaphoreType.DMA,
            pltpu.SemaphoreType.DMA,
            pltpu.SemaphoreType.DMA,
            pltpu.SemaphoreType.DMA,
            pltpu.SemaphoreType.DMA,
        ],
        compiler_params=pltpu.CompilerParams(needs_layout_passes=False),
    )
    def gather_k(tab_hbm, idx_hbm, out_hbm,
                 buf0, buf1, buf2, buf3, sem3, idx0, idx1, outv0, outv1,
                 sem0, sem1, sem2, sem_i0, sem_i1, sem_o0, sem_o1):
        w = lax.axis_index("c") * _NUM_SUBCORES + lax.axis_index("s")

        def third(r, off, n, buf, sem):
            return pltpu.make_async_copy(
                tab_hbm.at[r].at[pl.ds(off, n)], buf, sem)

        def idx_dma(k, buf, sem):
            return pltpu.make_async_copy(idx_hbm.at[k], buf, sem)

        def out_dma(r, buf, sem):
            return pltpu.make_async_copy(buf, out_hbm.at[r], sem)

        iota16 = lax.iota(jnp.int32, _LANES)

        def pass0(idx_v, out_v):
            def body(j, c2):
                for u in range(_UNROLL):
                    sl = pl.ds((j * _UNROLL + u) * _LANES, _LANES)
                    iv = idx_v[sl]
                    m = iv < _T0
                    out_v[sl] = plsc.load_gather(buf0, [iv], mask=m)
                return c2

            lax.fori_loop(0, _BVECS // _UNROLL, body, 0)

        def mid_pass(buf, off, size):
            def run(idx_v, out_v):
                def body(j, c2):
                    for u in range(_UNROLL):
                        v = j * _UNROLL + u
                        sl = pl.ds(v * _LANES, _LANES)
                        d = idx_v[sl] - off
                        m = d.astype(jnp.uint32) < jnp.uint32(size)
                        g = plsc.load_gather(buf, [d], mask=m)
                        out_v[sl] = jnp.where(m, g, out_v[sl])
                    return c2

                lax.fori_loop(0, _BVECS // _UNROLL, body, 0)

            return run

        def last_pass(buf, off):
            def run(idx_v, out_v):
                def body(j, c2):
                    for u in range(_UNROLL):
                        v = j * _UNROLL + u
                        sl = pl.ds(v * _LANES, _LANES)
                        d = idx_v[sl] - off
                        m = d >= 0
                        g = plsc.load_gather(buf, [d], mask=m)
                        out_v[sl] = jnp.where(m, g, out_v[sl])
                    return c2

                lax.fori_loop(0, _BVECS // _UNROLL, body, 0)

            return run

        pass1 = mid_pass(buf1, _OFF1, _T1)
        pass2 = mid_pass(buf2, _OFF2, _T2)
        pass3 = last_pass(buf3, _OFF3)

        def field(r, idx_v, out_v, has_next):
            for off, n, buf, sem, p in ((0, _T0, buf0, sem0, pass0),
                                        (_OFF1, _T1, buf1, sem1, pass1),
                                        (_OFF2, _T2, buf2, sem2, pass2),
                                        (_OFF3, _T3, buf3, sem3, pass3)):
                pltpu.make_async_copy(
                    tab_hbm.at[r].at[pl.ds(off, n)], buf, sem).wait()
                p(idx_v, out_v)

                @pl.when(has_next)
                def _(off=off, n=n, buf=buf, sem=sem):
                    third(r + _EMBED_DIM, off, n, buf, sem).start()

        third(w, 0, _T0, buf0, sem0).start()
        third(w, _OFF1, _T1, buf1, sem1).start()
        third(w, _OFF2, _T2, buf2, sem2).start()
        third(w, _OFF3, _T3, buf3, sem3).start()
        pltpu.sync_copy(idx_hbm.at[0], idx0)

        def field_pair(m, carry):
            k0 = m * 2
            k1 = k0 + 1
            r0 = k0 * _EMBED_DIM + w
            r1 = r0 + _EMBED_DIM

            @pl.when(m > 0)
            def _():
                out_dma(r0, outv0, sem_o0).wait()

            idx_dma(k1, idx1, sem_i1).start()
            field(r0, idx0, outv0, k1 < _NUM_FIELDS)
            out_dma(r0, outv0, sem_o0).start()

            @pl.when(m > 0)
            def _():
                out_dma(r1, outv1, sem_o1).wait()

            @pl.when(k1 + 1 < _NUM_FIELDS)
            def _():
                idx_dma(k1 + 1, idx0, sem_i0).start()

            idx_dma(k1, idx1, sem_i1).wait()
            field(r1, idx1, outv1, k1 + 1 < _NUM_FIELDS)
            out_dma(r1, outv1, sem_o1).start()

            @pl.when(k1 + 1 < _NUM_FIELDS)
            def _():
                idx_dma(k1 + 1, idx0, sem_i0).wait()

            return carry

        lax.fori_loop(0, _NUM_FIELDS // 2, field_pair, 0)
        out_dma(_ROWS - 2 * _EMBED_DIM + w, outv0, sem_o0).wait()
        out_dma(_ROWS - _EMBED_DIM + w, outv1, sem_o1).wait()

    return gather_k


_gather = _make_gather()


def kernel(x_cat, tables):
    tab2d = jnp.transpose(tables, (0, 2, 1)).reshape(_ROWS, _VOCAB)
    xt = jnp.transpose(x_cat.astype(jnp.int32))
    out_t = _gather(tab2d, xt)
    return jnp.transpose(out_t)


# speedup vs baseline: 1.7142x; 1.0140x over previous
import functools

import jax
import jax.numpy as jnp
from jax import lax
from jax.experimental import pallas as pl
from jax.experimental.pallas import tpu as pltpu
from jax.experimental.pallas import tpu_sc as plsc

_NUM_FIELDS = 26
_VOCAB = 100000
_T0 = 33408
_T1 = 33408
_T2 = _VOCAB - _T0 - _T1
_OFF1 = _T0
_OFF2 = _T0 + _T1
_EMBED_DIM = 32
_BATCH = 4096

_ROWS = _NUM_FIELDS * _EMBED_DIM
_NUM_CORES = 2
_NUM_SUBCORES = 16
_LANES = 16
_UNROLL = 16
_BVECS = _BATCH // _LANES


def _make_gather():
    mesh = plsc.VectorSubcoreMesh(core_axis_name="c", subcore_axis_name="s")

    @functools.partial(
        pl.kernel,
        mesh=mesh,
        out_type=jax.ShapeDtypeStruct((_ROWS, _BATCH), jnp.float32),
        scratch_types=[
            pltpu.VMEM((_T0,), jnp.float32),
            pltpu.VMEM((_T1,), jnp.float32),
            pltpu.VMEM((_T2,), jnp.float32),
            pltpu.VMEM((_BATCH,), jnp.int32),
            pltpu.VMEM((_BATCH,), jnp.int32),
            pltpu.VMEM((_BATCH,), jnp.float32),
            pltpu.VMEM((_BATCH,), jnp.float32),
            pltpu.SemaphoreType.DMA,
            pltpu.SemaphoreType.DMA,
            pltpu.SemaphoreType.DMA,
            pltpu.SemaphoreType.DMA,
            pltpu.SemaphoreType.DMA,
            pltpu.SemaphoreType.DMA,
            pltpu.SemaphoreType.DMA,
        ],
        compiler_params=pltpu.CompilerParams(needs_layout_passes=False),
    )
    def gather_k(tab_hbm, idx_hbm, out_hbm,
                 buf0, buf1, buf2, idx0, idx1, outv0, outv1,
                 sem0, sem1, sem2, sem_i0, sem_i1, sem_o0, sem_o1):
        w = lax.axis_index("c") * _NUM_SUBCORES + lax.axis_index("s")

        def third(r, off, n, buf, sem):
            return pltpu.make_async_copy(
                tab_hbm.at[r].at[pl.ds(off, n)], buf, sem)

        def idx_dma(k, buf, sem):
            return pltpu.make_async_copy(idx_hbm.at[k], buf, sem)

        def out_dma(r, buf, sem):
            return pltpu.make_async_copy(buf, out_hbm.at[r], sem)

        def pass0(idx_v, out_v):
            def body(j, c2):
                for u in range(_UNROLL):
                    sl = pl.ds((j * _UNROLL + u) * _LANES, _LANES)
                    iv = idx_v[sl]
                    m = iv < _T0
                    out_v[sl] = plsc.load_gather(buf0, [iv], mask=m)
                return c2

            lax.fori_loop(0, _BVECS // _UNROLL, body, 0)

        def pass1(idx_v, out_v):
            def body(j, c2):
                for u in range(_UNROLL):
                    sl = pl.ds((j * _UNROLL + u) * _LANES, _LANES)
                    d = idx_v[sl] - _OFF1
                    m = d.astype(jnp.uint32) < jnp.uint32(_T1)
                    g = plsc.load_gather(buf1, [d], mask=m)
                    out_v[sl] = jnp.where(m, g, out_v[sl])
                return c2

            lax.fori_loop(0, _BVECS // _UNROLL, body, 0)

        def pass2(idx_v, out_v):
            def body(j, c2):
                for u in range(_UNROLL):
                    sl = pl.ds((j * _UNROLL + u) * _LANES, _LANES)
                    d = idx_v[sl] - _OFF2
                    m = d >= 0
                    g = plsc.load_gather(buf2, [d], mask=m)
                    out_v[sl] = jnp.where(m, g, out_v[sl])
                return c2

            lax.fori_loop(0, _BVECS // _UNROLL, body, 0)

        def field(r, idx_v, out_v, has_next):
            pltpu.make_async_copy(
                tab_hbm.at[r].at[pl.ds(0, _T0)], buf0, sem0).wait()
            pass0(idx_v, out_v)

            @pl.when(has_next)
            def _():
                third(r + _EMBED_DIM, 0, _T0, buf0, sem0).start()

            pltpu.make_async_copy(
                tab_hbm.at[r].at[pl.ds(_OFF1, _T1)], buf1, sem1).wait()
            pass1(idx_v, out_v)

            @pl.when(has_next)
            def _():
                third(r + _EMBED_DIM, _OFF1, _T1, buf1, sem1).start()

            pltpu.make_async_copy(
                tab_hbm.at[r].at[pl.ds(_OFF2, _T2)], buf2, sem2).wait()
            pass2(idx_v, out_v)

            @pl.when(has_next)
            def _():
                third(r + _EMBED_DIM, _OFF2, _T2, buf2, sem2).start()

        third(w, 0, _T0, buf0, sem0).start()
        third(w, _OFF1, _T1, buf1, sem1).start()
        third(w, _OFF2, _T2, buf2, sem2).start()
        pltpu.sync_copy(idx_hbm.at[0], idx0)

        def field_pair(m, carry):
            k0 = m * 2
            k1 = k0 + 1
            r0 = k0 * _EMBED_DIM + w
            r1 = r0 + _EMBED_DIM

            @pl.when(m > 0)
            def _():
                out_dma(r0, outv0, sem_o0).wait()

            idx_dma(k1, idx1, sem_i1).start()
            field(r0, idx0, outv0, k1 < _NUM_FIELDS)
            out_dma(r0, outv0, sem_o0).start()

            @pl.when(m > 0)
            def _():
                out_dma(r1, outv1, sem_o1).wait()

            @pl.when(k1 + 1 < _NUM_FIELDS)
            def _():
                idx_dma(k1 + 1, idx0, sem_i0).start()

            idx_dma(k1, idx1, sem_i1).wait()
            field(r1, idx1, outv1, k1 + 1 < _NUM_FIELDS)
            out_dma(r1, outv1, sem_o1).start()

            @pl.when(k1 + 1 < _NUM_FIELDS)
            def _():
                idx_dma(k1 + 1, idx0, sem_i0).wait()

            return carry

        lax.fori_loop(0, _NUM_FIELDS // 2, field_pair, 0)
        out_dma(_ROWS - 2 * _EMBED_DIM + w, outv0, sem_o0).wait()
        out_dma(_ROWS - _EMBED_DIM + w, outv1, sem_o1).wait()

    return gather_k


_gather = _make_gather()


def kernel(x_cat, tables):
    tab2d = jnp.transpose(tables, (0, 2, 1)).reshape(_ROWS, _VOCAB)
    xt = jnp.transpose(x_cat.astype(jnp.int32))
    out_t = _gather(tab2d, xt)
    return jnp.transpose(out_t)


# speedup vs baseline: 1.7302x; 1.0093x over previous
import functools

import jax
import jax.numpy as jnp
from jax import lax
from jax.experimental import pallas as pl
from jax.experimental.pallas import tpu as pltpu
from jax.experimental.pallas import tpu_sc as plsc

_NUM_FIELDS = 26
_VOCAB = 100000
_T0 = 33408
_T1 = 33408
_T2 = _VOCAB - _T0 - _T1
_OFF1 = _T0
_OFF2 = _T0 + _T1
_EMBED_DIM = 32
_BATCH = 4096

_ROWS = _NUM_FIELDS * _EMBED_DIM
_NUM_CORES = 2
_NUM_SUBCORES = 16
_LANES = 16
_UNROLL = 8
_BVECS = _BATCH // _LANES


def _make_gather():
    mesh = plsc.VectorSubcoreMesh(core_axis_name="c", subcore_axis_name="s")

    @functools.partial(
        pl.kernel,
        mesh=mesh,
        out_type=jax.ShapeDtypeStruct((_ROWS, _BATCH), jnp.float32),
        scratch_types=[
            pltpu.VMEM((_T0,), jnp.float32),
            pltpu.VMEM((_T1,), jnp.float32),
            pltpu.VMEM((_T2,), jnp.float32),
            pltpu.VMEM((_BATCH,), jnp.int32),
            pltpu.VMEM((_BATCH,), jnp.int32),
            pltpu.VMEM((_BATCH,), jnp.float32),
            pltpu.VMEM((_BATCH,), jnp.float32),
            pltpu.SemaphoreType.DMA,
            pltpu.SemaphoreType.DMA,
            pltpu.SemaphoreType.DMA,
            pltpu.SemaphoreType.DMA,
            pltpu.SemaphoreType.DMA,
            pltpu.SemaphoreType.DMA,
            pltpu.SemaphoreType.DMA,
        ],
        compiler_params=pltpu.CompilerParams(needs_layout_passes=False),
    )
    def gather_k(tab_hbm, idx_hbm, out_hbm,
                 buf0, buf1, buf2, idx0, idx1, outv0, outv1,
                 sem0, sem1, sem2, sem_i0, sem_i1, sem_o0, sem_o1):
        w = lax.axis_index("c") * _NUM_SUBCORES + lax.axis_index("s")

        def third(r, off, n, buf, sem):
            return pltpu.make_async_copy(
                tab_hbm.at[r].at[pl.ds(off, n)], buf, sem)

        def idx_dma(k, buf, sem):
            return pltpu.make_async_copy(idx_hbm.at[k], buf, sem)

        def out_dma(r, buf, sem):
            return pltpu.make_async_copy(buf, out_hbm.at[r], sem)

        def pass0(idx_v, out_v):
            def body(j, c2):
                for u in range(_UNROLL):
                    sl = pl.ds((j * _UNROLL + u) * _LANES, _LANES)
                    iv = idx_v[sl]
                    m = iv < _T0
                    out_v[sl] = plsc.load_gather(buf0, [iv], mask=m)
                return c2

            lax.fori_loop(0, _BVECS // _UNROLL, body, 0)

        def pass1(idx_v, out_v):
            def body(j, c2):
                for u in range(_UNROLL):
                    sl = pl.ds((j * _UNROLL + u) * _LANES, _LANES)
                    d = idx_v[sl] - _OFF1
                    m = d.astype(jnp.uint32) < jnp.uint32(_T1)
                    g = plsc.load_gather(buf1, [d], mask=m)
                    out_v[sl] = jnp.where(m, g, out_v[sl])
                return c2

            lax.fori_loop(0, _BVECS // _UNROLL, body, 0)

        def pass2(idx_v, out_v):
            def body(j, c2):
                for u in range(_UNROLL):
                    sl = pl.ds((j * _UNROLL + u) * _LANES, _LANES)
                    d = idx_v[sl] - _OFF2
                    m = d >= 0
                    g = plsc.load_gather(buf2, [d], mask=m)
                    out_v[sl] = jnp.where(m, g, out_v[sl])
                return c2

            lax.fori_loop(0, _BVECS // _UNROLL, body, 0)

        def field(r, idx_v, out_v, has_next):
            pltpu.make_async_copy(
                tab_hbm.at[r].at[pl.ds(0, _T0)], buf0, sem0).wait()
            pass0(idx_v, out_v)

            @pl.when(has_next)
            def _():
                third(r + _EMBED_DIM, 0, _T0, buf0, sem0).start()

            pltpu.make_async_copy(
                tab_hbm.at[r].at[pl.ds(_OFF1, _T1)], buf1, sem1).wait()
            pass1(idx_v, out_v)

            @pl.when(has_next)
            def _():
                third(r + _EMBED_DIM, _OFF1, _T1, buf1, sem1).start()

            pltpu.make_async_copy(
                tab_hbm.at[r].at[pl.ds(_OFF2, _T2)], buf2, sem2).wait()
            pass2(idx_v, out_v)

            @pl.when(has_next)
            def _():
                third(r + _EMBED_DIM, _OFF2, _T2, buf2, sem2).start()

        third(w, 0, _T0, buf0, sem0).start()
        third(w, _OFF1, _T1, buf1, sem1).start()
        third(w, _OFF2, _T2, buf2, sem2).start()
        pltpu.sync_copy(idx_hbm.at[0], idx0)

        def field_pair(m, carry):
            k0 = m * 2
            k1 = k0 + 1
            r0 = k0 * _EMBED_DIM + w
            r1 = r0 + _EMBED_DIM

            @pl.when(m > 0)
            def _():
                out_dma(r0, outv0, sem_o0).wait()

            idx_dma(k1, idx1, sem_i1).start()
            field(r0, idx0, outv0, k1 < _NUM_FIELDS)
            out_dma(r0, outv0, sem_o0).start()

            @pl.when(m > 0)
            def _():
                out_dma(r1, outv1, sem_o1).wait()

            @pl.when(k1 + 1 < _NUM_FIELDS)
            def _():
                idx_dma(k1 + 1, idx0, sem_i0).start()

            idx_dma(k1, idx1, sem_i1).wait()
            field(r1, idx1, outv1, k1 + 1 < _NUM_FIELDS)
            out_dma(r1, outv1, sem_o1).start()

            @pl.when(k1 + 1 < _NUM_FIELDS)
            def _():
                idx_dma(k1 + 1, idx0, sem_i0).wait()

            return carry

        lax.fori_loop(0, _NUM_FIELDS // 2, field_pair, 0)
        out_dma(_ROWS - 2 * _EMBED_DIM + w, outv0, sem_o0).wait()
        out_dma(_ROWS - _EMBED_DIM + w, outv1, sem_o1).wait()

    return gather_k


_gather = _make_gather()


def kernel(x_cat, tables):
    tab2d = jnp.transpose(tables, (0, 2, 1)).reshape(_ROWS, _VOCAB)
    xt = jnp.transpose(x_cat.astype(jnp.int32))
    out_t = _gather(tab2d, xt)
    return jnp.transpose(out_t)
